# baseline (speedup 1.0000x reference)
import functools

import jax
import jax.numpy as jnp
from jax import lax
from jax.experimental import pallas as pl
from jax.experimental.pallas import tpu as pltpu
from jax.experimental.pallas import tpu_sc as plsc

B = 16384
CONT = 13
EMB = 64
NCAT = 26
NDENSE = CONT + 1
DOUT = (NDENSE + NCAT) * EMB
DCOL = NDENSE * EMB
CCOL = NCAT * EMB

NC = 2
NS = 16
NW = NC * NS
R = 16
IDX_PER_BLK = R * NCAT
GCH = 4
IPG = IDX_PER_BLK // GCH
NSLOT = 4

NCHUNK = 2
BCH = B // NCHUNK
BM = 256
TILES = BCH // BM

ROWS_PER_W = BCH // NW
GROUPS = ROWS_PER_W // R
IDX_ROWS = ROWS_PER_W * NCAT // IPG


def _sc_body(idx_hbm, tab_hbm, out_hbm, idx_v, rows_v, *sems):
    wid = lax.axis_index("s") * NC + lax.axis_index("c")
    sem_g = sems[:NSLOT]
    sem_o = sems[NSLOT:]

    pltpu.sync_copy(idx_hbm.at[pl.ds(wid * IDX_ROWS, IDX_ROWS)], idx_v)

    def gather_start(blk, s):
        for i in range(GCH):
            pltpu.async_copy(
                tab_hbm.at[idx_v.at[blk * GCH + i]],
                rows_v.at[s, pl.ds(i * IPG, IPG)],
                sem_g[s])

    def gather_wait(s):
        for i in range(GCH):
            pltpu.make_async_copy(
                tab_hbm.at[idx_v.at[0]],
                rows_v.at[s, pl.ds(i * IPG, IPG)],
                sem_g[s]).wait()

    def out_start(blk, s):
        pltpu.async_copy(
            rows_v.at[s],
            out_hbm.at[pl.ds(wid * ROWS_PER_W * NCAT + blk * IDX_PER_BLK,
                             IDX_PER_BLK)],
            sem_o[s])

    def out_wait(s):
        pltpu.make_async_copy(
            rows_v.at[s],
            out_hbm.at[pl.ds(0, IDX_PER_BLK)],
            sem_o[s]).wait()

    for b in range(NSLOT):
        gather_start(b, b)

    def ring_body(i, carry):
        for b in range(NSLOT):
            blk = NSLOT * i + b
            gather_wait(b)
            out_start(blk, b)
            s2 = (b + 2) % NSLOT

            def prefetch(blk=blk, s2=s2):
                out_wait(s2)
                gather_start(blk + 2, s2)

            pl.when(jnp.logical_and(blk >= 2, blk + 2 < GROUPS))(prefetch)
        return carry

    lax.fori_loop(0, GROUPS // NSLOT, ring_body, 0)
    for b in range(NSLOT):
        out_wait(b)


@functools.partial(
    pl.kernel,
    out_type=jax.ShapeDtypeStruct((BCH * NCAT, EMB), jnp.float32),
    mesh=plsc.VectorSubcoreMesh(core_axis_name="c", subcore_axis_name="s"),
    compiler_params=pltpu.CompilerParams(use_tc_tiling_on_sc=False),
    scratch_types=[
        pltpu.VMEM((IDX_ROWS, IPG), jnp.int32),
        pltpu.VMEM((NSLOT, IDX_PER_BLK, EMB), jnp.float32),
    ] + [pltpu.SemaphoreType.DMA] * (2 * NSLOT),
)
def _gather_sc(idx_hbm, tab_hbm, out_hbm, idx_v, rows_v, *sems):
    _sc_body(idx_hbm, tab_hbm, out_hbm, idx_v, rows_v, *sems)


def _assemble_tc(xc_ref, cat_ref, wf_ref, bf_ref, bc_ref, out_ref):
    col = lax.broadcasted_iota(jnp.int32, (16, DCOL), 1) // EMB
    row = lax.broadcasted_iota(jnp.int32, (16, DCOL), 0)
    sel = (col == row).astype(jnp.float32)
    xrep = jnp.dot(xc_ref[...], sel, preferred_element_type=jnp.float32)
    out_ref[:, :DCOL] = xrep * wf_ref[...] + bf_ref[...]
    out_ref[:, DCOL:] = cat_ref[...] + bc_ref[...]


def _assemble_tc_alias(prev_ref, xc_ref, cat_ref, wf_ref, bf_ref, bc_ref,
                       out_ref):
    _assemble_tc(xc_ref, cat_ref, wf_ref, bf_ref, bc_ref, out_ref)


def _tc_chunk(c, prev, xc_pad, cat_rows, wf, bf, bc):
    specs = [
        pl.BlockSpec((BM, 16), lambda i, c=c: (c * TILES + i, 0)),
        pl.BlockSpec((BM, CCOL), lambda i: (i, 0)),
        pl.BlockSpec((1, DCOL), lambda i: (0, 0)),
        pl.BlockSpec((1, DCOL), lambda i: (0, 0)),
        pl.BlockSpec((1, CCOL), lambda i: (0, 0)),
    ]
    out_spec = pl.BlockSpec((BM, DOUT), lambda i, c=c: (c * TILES + i, 0))
    out_shape = jax.ShapeDtypeStruct((B, DOUT), jnp.float32)
    if prev is None:
        return pl.pallas_call(
            _assemble_tc,
            grid=(TILES,),
            in_specs=specs,
            out_specs=out_spec,
            out_shape=out_shape,
        )(xc_pad, cat_rows, wf, bf, bc)
    return pl.pallas_call(
        _assemble_tc_alias,
        grid=(TILES,),
        in_specs=[pl.BlockSpec(memory_space=pltpu.MemorySpace.HBM)] + specs,
        out_specs=out_spec,
        out_shape=out_shape,
        input_output_aliases={0: 0},
    )(prev, xc_pad, cat_rows, wf, bf, bc)


def kernel(x_conts, x_cats, weight, bias, cat_table, category_offsets):
    flat_idx = (x_cats.astype(jnp.int32)
                + category_offsets.astype(jnp.int32)[None, :])
    flat_idx = flat_idx.reshape(NCHUNK, BCH * NCAT // IPG, IPG)

    xc_pad = jnp.zeros((B, 16), jnp.float32)
    xc_pad = xc_pad.at[:, 0].set(1.0).at[:, 1:NDENSE].set(x_conts)
    wf = weight.reshape(1, DCOL)
    bf = jnp.concatenate(
        [jnp.zeros((1, EMB), jnp.float32), bias[:CONT].reshape(1, CONT * EMB)],
        axis=1)
    bc = bias[CONT:].reshape(1, CCOL)

    cat_chunks = [
        _gather_sc(flat_idx[c], cat_table).reshape(BCH, CCOL)
        for c in range(NCHUNK)
    ]
    out = None
    for c in range(NCHUNK):
        out = _tc_chunk(c, out, xc_pad, cat_chunks[c], wf, bf, bc)
    return out


# speedup vs baseline: 1.0396x; 1.0396x over previous
import functools

import jax
import jax.numpy as jnp
from jax import lax
from jax.experimental import pallas as pl
from jax.experimental.pallas import tpu as pltpu
from jax.experimental.pallas import tpu_sc as plsc

B = 16384
CONT = 13
EMB = 64
NCAT = 26
NDENSE = CONT + 1
DOUT = (NDENSE + NCAT) * EMB
DCOL = NDENSE * EMB
CCOL = NCAT * EMB

NC = 2
NS = 16
NW = NC * NS
R = 16
IDX_PER_BLK = R * NCAT
GCH = 4
IPG = IDX_PER_BLK // GCH
NSLOT = 4

NCHUNK = 1
BCH = B // NCHUNK
BM = 512
TILES = BCH // BM

ROWS_PER_W = BCH // NW
GROUPS = ROWS_PER_W // R
IDX_ROWS = ROWS_PER_W * NCAT // IPG


def _sc_body(idx_hbm, tab_hbm, out_hbm, idx_v, rows_v, *sems):
    wid = lax.axis_index("s") * NC + lax.axis_index("c")
    sem_g = sems[:NSLOT]
    sem_o = sems[NSLOT:]

    pltpu.sync_copy(idx_hbm.at[pl.ds(wid * IDX_ROWS, IDX_ROWS)], idx_v)

    def gather_start(blk, s):
        for i in range(GCH):
            pltpu.async_copy(
                tab_hbm.at[idx_v.at[blk * GCH + i]],
                rows_v.at[s, pl.ds(i * IPG, IPG)],
                sem_g[s])

    def gather_wait(s):
        for i in range(GCH):
            pltpu.make_async_copy(
                tab_hbm.at[idx_v.at[0]],
                rows_v.at[s, pl.ds(i * IPG, IPG)],
                sem_g[s]).wait()

    def out_start(blk, s):
        pltpu.async_copy(
            rows_v.at[s],
            out_hbm.at[pl.ds(wid * ROWS_PER_W * NCAT + blk * IDX_PER_BLK,
                             IDX_PER_BLK)],
            sem_o[s])

    def out_wait(s):
        pltpu.make_async_copy(
            rows_v.at[s],
            out_hbm.at[pl.ds(0, IDX_PER_BLK)],
            sem_o[s]).wait()

    for b in range(NSLOT):
        gather_start(b, b)

    def ring_body(i, carry):
        for b in range(NSLOT):
            blk = NSLOT * i + b
            gather_wait(b)
            out_start(blk, b)
            s2 = (b + 2) % NSLOT

            def prefetch(blk=blk, s2=s2):
                out_wait(s2)
                gather_start(blk + 2, s2)

            pl.when(jnp.logical_and(blk >= 2, blk + 2 < GROUPS))(prefetch)
        return carry

    lax.fori_loop(0, GROUPS // NSLOT, ring_body, 0)
    for b in range(NSLOT):
        out_wait(b)


@functools.partial(
    pl.kernel,
    out_type=jax.ShapeDtypeStruct((BCH * NCAT, EMB), jnp.float32),
    mesh=plsc.VectorSubcoreMesh(core_axis_name="c", subcore_axis_name="s"),
    compiler_params=pltpu.CompilerParams(use_tc_tiling_on_sc=False),
    scratch_types=[
        pltpu.VMEM((IDX_ROWS, IPG), jnp.int32),
        pltpu.VMEM((NSLOT, IDX_PER_BLK, EMB), jnp.float32),
    ] + [pltpu.SemaphoreType.DMA] * (2 * NSLOT),
)
def _gather_sc(idx_hbm, tab_hbm, out_hbm, idx_v, rows_v, *sems):
    _sc_body(idx_hbm, tab_hbm, out_hbm, idx_v, rows_v, *sems)


def _assemble_tc(xc_ref, cat_ref, wf_ref, bf_ref, bc_ref, out_ref):
    col = lax.broadcasted_iota(jnp.int32, (16, DCOL), 1) // EMB
    row = lax.broadcasted_iota(jnp.int32, (16, DCOL), 0)
    sel = (col == row).astype(jnp.float32)
    xrep = jnp.dot(xc_ref[...], sel, preferred_element_type=jnp.float32)
    out_ref[:, :DCOL] = xrep * wf_ref[...] + bf_ref[...]
    out_ref[:, DCOL:] = cat_ref[...] + bc_ref[...]


def _assemble_tc_alias(prev_ref, xc_ref, cat_ref, wf_ref, bf_ref, bc_ref,
                       out_ref):
    _assemble_tc(xc_ref, cat_ref, wf_ref, bf_ref, bc_ref, out_ref)


def _tc_chunk(c, prev, xc_pad, cat_rows, wf, bf, bc):
    specs = [
        pl.BlockSpec((BM, 16), lambda i, c=c: (c * TILES + i, 0)),
        pl.BlockSpec((BM, CCOL), lambda i: (i, 0)),
        pl.BlockSpec((1, DCOL), lambda i: (0, 0)),
        pl.BlockSpec((1, DCOL), lambda i: (0, 0)),
        pl.BlockSpec((1, CCOL), lambda i: (0, 0)),
    ]
    out_spec = pl.BlockSpec((BM, DOUT), lambda i, c=c: (c * TILES + i, 0))
    out_shape = jax.ShapeDtypeStruct((B, DOUT), jnp.float32)
    if prev is None:
        return pl.pallas_call(
            _assemble_tc,
            grid=(TILES,),
            in_specs=specs,
            out_specs=out_spec,
            out_shape=out_shape,
        )(xc_pad, cat_rows, wf, bf, bc)
    return pl.pallas_call(
        _assemble_tc_alias,
        grid=(TILES,),
        in_specs=[pl.BlockSpec(memory_space=pltpu.MemorySpace.HBM)] + specs,
        out_specs=out_spec,
        out_shape=out_shape,
        input_output_aliases={0: 0},
    )(prev, xc_pad, cat_rows, wf, bf, bc)


def kernel(x_conts, x_cats, weight, bias, cat_table, category_offsets):
    flat_idx = (x_cats.astype(jnp.int32)
                + category_offsets.astype(jnp.int32)[None, :])
    flat_idx = flat_idx.reshape(NCHUNK, BCH * NCAT // IPG, IPG)

    xc_pad = jnp.zeros((B, 16), jnp.float32)
    xc_pad = xc_pad.at[:, 0].set(1.0).at[:, 1:NDENSE].set(x_conts)
    wf = weight.reshape(1, DCOL)
    bf = jnp.concatenate(
        [jnp.zeros((1, EMB), jnp.float32), bias[:CONT].reshape(1, CONT * EMB)],
        axis=1)
    bc = bias[CONT:].reshape(1, CCOL)

    cat_chunks = [
        _gather_sc(flat_idx[c], cat_table).reshape(BCH, CCOL)
        for c in range(NCHUNK)
    ]
    out = None
    for c in range(NCHUNK):
        out = _tc_chunk(c, out, xc_pad, cat_chunks[c], wf, bf, bc)
    return out
